# Initial kernel scaffold; baseline (speedup 1.0000x reference)
import jax
import jax.numpy as jnp
from jax.experimental import pallas as pl


def kernel(vid, params):
    raise NotImplementedError("write your pallas kernel here")


# speedup vs baseline: 10.4021x; 10.4021x over previous
import jax
import jax.numpy as jnp
from jax.experimental import pallas as pl
from jax.experimental.pallas import tpu as pltpu

C = 96
H = 224
W = 224
KTOP = 10
STRIDE = 14
NKEY = (H // STRIDE) * (W // STRIDE)
RSCALE = 1.0 / (C ** 0.5)

_MR = 8
_CR = 8


def _dot16(a, b):
    return jnp.dot(a.astype(jnp.bfloat16), b.astype(jnp.bfloat16),
                   preferred_element_type=jnp.float32)


def _merge_kernel(x_ref, kf_ref, wq_ref, wk_ref, wv_ref, wo_ref, o_ref,
                  kkt_s, vv_s):
    @pl.when(pl.program_id(0) == 0)
    def _():
        kf = kf_ref[...]
        kkt_s[...] = _dot16(kf, wk_ref[...]).T
        vv_s[...] = _dot16(kf, wv_ref[...])

    xb = x_ref[...].reshape(_MR * W, C)
    q = _dot16(xb, wq_ref[...])
    sim = _dot16(q, kkt_s[...]) / jnp.sqrt(jnp.float32(C))
    m1 = jnp.max(sim, axis=-1, keepdims=True)
    cur = sim
    m = m1
    for _ in range(KTOP - 1):
        cur = jnp.where(cur >= m, -jnp.inf, cur)
        m = jnp.max(cur, axis=-1, keepdims=True)
    p = jnp.where(sim >= m, jnp.exp(sim - m1), 0.0)
    denom = jnp.sum(p, axis=-1, keepdims=True)
    t = jnp.dot(p, vv_s[...], preferred_element_type=jnp.float32,
                precision=jax.lax.Precision.HIGHEST) / denom
    o = _dot16(t, wo_ref[...])
    o_ref[...] = (xb + o).reshape(_MR, W, C)


def _merge(x, kf, p):
    return pl.pallas_call(
        _merge_kernel,
        grid=(H // _MR,),
        in_specs=[
            pl.BlockSpec((_MR, W, C), lambda i: (i, 0, 0)),
            pl.BlockSpec((NKEY, C), lambda i: (0, 0)),
            pl.BlockSpec((C, C), lambda i: (0, 0)),
            pl.BlockSpec((C, C), lambda i: (0, 0)),
            pl.BlockSpec((C, C), lambda i: (0, 0)),
            pl.BlockSpec((C, C), lambda i: (0, 0)),
        ],
        out_specs=pl.BlockSpec((_MR, W, C), lambda i: (i, 0, 0)),
        out_shape=jax.ShapeDtypeStruct((H, W, C), jnp.float32),
        scratch_shapes=[
            pltpu.VMEM((C, NKEY), jnp.float32),
            pltpu.VMEM((NKEY, C), jnp.float32),
        ],
    )(x, kf, p['Wq'], p['Wk'], p['Wv'], p['Wo'])


def _rb_kernel(xp_ref, w1_ref, b1_ref, a_ref, w2_ref, b2_ref, o_ref):
    base = pl.program_id(0) * _CR
    R = _CR
    acc = jnp.zeros((R + 2, W + 2, C), jnp.float32)
    for dy in range(3):
        slab = xp_ref[pl.ds(base + dy, R + 2), :, :]
        s2 = slab.reshape((R + 2) * (W + 4), C)
        for dx in range(3):
            sm = _dot16(s2, w1_ref[dy, dx])
            acc = acc + sm.reshape(R + 2, W + 4, C)[:, dx:dx + W + 2, :]
    hb = acc + b1_ref[...]
    jrow = jax.lax.broadcasted_iota(jnp.int32, (R + 2, W + 2, C), 0)
    jcol = jax.lax.broadcasted_iota(jnp.int32, (R + 2, W + 2, C), 1)
    tr = jrow + (base - 1)
    valid = (tr >= 0) & (tr < H) & (jcol >= 1) & (jcol <= W)
    hb = jnp.where(valid, hb, 0.0)
    alpha = a_ref[0, 0]
    hb = jnp.where(hb > 0, hb, alpha * hb)
    acc2 = jnp.zeros((R, W, C), jnp.float32)
    for dy in range(3):
        hs = hb[dy:dy + R].reshape(R * (W + 2), C)
        for dx in range(3):
            sm2 = _dot16(hs, w2_ref[dy, dx])
            acc2 = acc2 + sm2.reshape(R, W + 2, C)[:, dx:dx + W, :]
    xres = xp_ref[pl.ds(base + 2, R), 2:2 + W, :]
    o_ref[...] = acc2 + b2_ref[...] + xres


def _resblock(x, p):
    xp = jnp.pad(x, ((2, 2), (2, 2), (0, 0)))
    w1t = jnp.transpose(p['w1'], (2, 3, 1, 0))
    w2t = jnp.transpose(p['w2'], (2, 3, 1, 0))
    return pl.pallas_call(
        _rb_kernel,
        grid=(H // _CR,),
        in_specs=[
            pl.BlockSpec((H + 4, W + 4, C), lambda i: (0, 0, 0)),
            pl.BlockSpec((3, 3, C, C), lambda i: (0, 0, 0, 0)),
            pl.BlockSpec((1, C), lambda i: (0, 0)),
            pl.BlockSpec((1, 1), lambda i: (0, 0)),
            pl.BlockSpec((3, 3, C, C), lambda i: (0, 0, 0, 0)),
            pl.BlockSpec((1, C), lambda i: (0, 0)),
        ],
        out_specs=pl.BlockSpec((_CR, W, C), lambda i: (i, 0, 0)),
        out_shape=jax.ShapeDtypeStruct((H, W, C), jnp.float32),
    )(xp, w1t, p['b1'][None, :], p['a'].reshape(1, 1), w2t, p['b2'][None, :])


def kernel(vid, params):
    x = jnp.transpose(vid[0], (1, 2, 0))
    for mname, rname in (('c1', 'rbs1'), ('c2', 'rbs2'), ('c3', None)):
        kf = x[::STRIDE, ::STRIDE, :].reshape(NKEY, C)
        x = _merge(x, kf, params[mname])
        if rname is not None:
            x = _resblock(x, params[rname])
    return jnp.transpose(x, (2, 0, 1))[None]
